# baseline (speedup 1.0000x reference)
import functools

import jax
import jax.numpy as jnp
from jax import lax
from jax.experimental import pallas as pl
from jax.experimental.pallas import tpu as pltpu
from jax.experimental.pallas import tpu_sc as plsc

EMBED = 16
L = 16
NC = 2
NS = 16
NW = NC * NS
CHUNK = 128
ROWS_PER_SUPER = 8


def _build_sc_call(B):
  b_per_w = B // NW
  n_chunks = b_per_w // CHUNK
  groups_per_chunk = CHUNK // L
  mesh = plsc.VectorSubcoreMesh(
      core_axis_name="c", subcore_axis_name="s",
      num_cores=NC, num_subcores=NS)

  @functools.partial(
      pl.kernel,
      out_type=jax.ShapeDtypeStruct((B,), jnp.float32),
      mesh=mesh,
      compiler_params=pltpu.CompilerParams(
          needs_layout_passes=False, use_tc_tiling_on_sc=True),
      scratch_types=[
          pltpu.VMEM((2 * n_chunks, CHUNK), jnp.int32),
          pltpu.VMEM((2 * n_chunks, CHUNK), jnp.int32),
          pltpu.VMEM((CHUNK, 8 * EMBED), jnp.float32),
          pltpu.VMEM((CHUNK, 8 * EMBED), jnp.float32),
          pltpu.VMEM((b_per_w,), jnp.float32),
          pltpu.VMEM((EMBED, L), jnp.float32),
          pltpu.VMEM((L,), jnp.float32),
          pltpu.SemaphoreType.DMA,
          pltpu.SemaphoreType.DMA,
      ],
  )
  def gmf(user_h, item_h, u_tab, i_tab, w_h, b_h, out_h,
          idxo, sup, ubuf, ibuf, outv, wv, bv, usem, isem):
    wid = lax.axis_index("s") * NC + lax.axis_index("c")
    base4 = pl.multiple_of(wid * n_chunks, n_chunks)

    pltpu.sync_copy(user_h.at[pl.ds(base4, n_chunks)],
                    idxo.at[pl.ds(0, n_chunks)])
    pltpu.sync_copy(item_h.at[pl.ds(base4, n_chunks)],
                    idxo.at[pl.ds(n_chunks, n_chunks)])
    pltpu.sync_copy(w_h, wv)
    pltpu.sync_copy(b_h, bv)

    for r in range(2 * n_chunks):
      for k in range(CHUNK // L):
        v = idxo[r, pl.ds(k * L, L)]
        sup[r, pl.ds(k * L, L)] = lax.shift_right_logical(v, 3)

    bias_vec = bv[...]
    wsp = [wv[e] for e in range(EMBED)]
    iot = lax.iota(jnp.int32, L)
    zero = jnp.zeros((L,), jnp.float32)
    seven = jnp.full((L,), 7, jnp.int32)

    for j in range(n_chunks):
      cpu = pltpu.async_copy(u_tab.at[sup.at[j]], ubuf, usem)
      cpi = pltpu.async_copy(i_tab.at[sup.at[n_chunks + j]], ibuf, isem)
      cpu.wait()
      cpi.wait()
      for k in range(groups_per_chunk):
        ou = idxo[j, pl.ds(k * L, L)]
        oi = idxo[n_chunks + j, pl.ds(k * L, L)]
        cbu = lax.shift_left(lax.bitwise_and(ou, seven), 4)
        cbi = lax.shift_left(lax.bitwise_and(oi, seven), 4)
        rows = iot + (k * L)
        accs = [bias_vec, zero, zero, zero]
        for e in range(EMBED):
          uc = plsc.load_gather(ubuf, [rows, cbu + e])
          ic = plsc.load_gather(ibuf, [rows, cbi + e])
          accs[e % 4] = accs[e % 4] + (uc * ic) * wsp[e]
        outv[pl.ds(j * CHUNK + k * L, L)] = (
            (accs[0] + accs[1]) + (accs[2] + accs[3]))

    base = pl.multiple_of(wid * b_per_w, b_per_w)
    pltpu.sync_copy(outv, out_h.at[pl.ds(base, b_per_w)])

  return gmf


def kernel(user, item, U, I, fc_w, fc_b):
  B = user.shape[0]
  n_rows = U.shape[0]
  u_tab = U.reshape(n_rows // ROWS_PER_SUPER, ROWS_PER_SUPER * EMBED)
  i_tab = I.reshape(I.shape[0] // ROWS_PER_SUPER, ROWS_PER_SUPER * EMBED)
  user2 = user.astype(jnp.int32).reshape(B // CHUNK, CHUNK)
  item2 = item.astype(jnp.int32).reshape(B // CHUNK, CHUNK)
  w_vec = jnp.broadcast_to(
      fc_w.reshape(EMBED, 1).astype(jnp.float32), (EMBED, L))
  b_vec = jnp.broadcast_to(fc_b.reshape(()), (L,)).astype(jnp.float32)
  return _build_sc_call(B)(user2, item2, u_tab, i_tab, w_vec, b_vec)


# speedup vs baseline: 5.5915x; 5.5915x over previous
import functools

import jax
import jax.numpy as jnp
from jax import lax
from jax.experimental import pallas as pl
from jax.experimental.pallas import tpu as pltpu
from jax.experimental.pallas import tpu_sc as plsc

EMBED = 16
L = 16
NC = 2
NS = 16
NW = NC * NS
CH = 32
BLK = 128


def _build_sc_call(B):
  b_per_w = B // NW
  n_rounds = b_per_w // CH
  n_groups = b_per_w // L
  mesh = plsc.VectorSubcoreMesh(
      core_axis_name="c", subcore_axis_name="s",
      num_cores=NC, num_subcores=NS)

  @functools.partial(
      pl.kernel,
      out_type=jax.ShapeDtypeStruct((B,), jnp.float32),
      mesh=mesh,
      compiler_params=pltpu.CompilerParams(
          needs_layout_passes=False, use_tc_tiling_on_sc=True),
      scratch_types=[
          pltpu.VMEM((b_per_w,), jnp.int32),
          pltpu.VMEM((b_per_w,), jnp.int32),
          pltpu.VMEM((CH * EMBED, BLK), jnp.float32),
          pltpu.VMEM((b_per_w * EMBED,), jnp.float32),
          pltpu.VMEM((b_per_w * EMBED,), jnp.float32),
          pltpu.VMEM((b_per_w,), jnp.float32),
          pltpu.VMEM((EMBED, L), jnp.float32),
          pltpu.VMEM((L,), jnp.float32),
          pltpu.SemaphoreType.DMA,
      ],
  )
  def gmf(user_h, item_h, u_t, i_t, w_h, b_h, out_h,
          uidx, iidx, blk, urows, irows, outv, wv, bv, sem):
    wid = lax.axis_index("s") * NC + lax.axis_index("c")
    base = pl.multiple_of(wid * b_per_w, b_per_w)

    pltpu.sync_copy(user_h.at[pl.ds(base, b_per_w)], uidx)
    pltpu.sync_copy(item_h.at[pl.ds(base, b_per_w)], iidx)
    pltpu.sync_copy(w_h, wv)
    pltpu.sync_copy(b_h, bv)

    iot = lax.iota(jnp.int32, L)
    lanes_mask = jnp.full((L,), BLK - 1, jnp.int32)

    def fetch_round(tab, idx_ref, rows_ref, c):
      c0 = pl.multiple_of(c * CH, CH)
      vecs = [idx_ref[pl.ds(c0 + v * L, L)] for v in range(CH // L)]
      copies = []
      for j in range(CH):
        r = vecs[j // L][j % L]
        col0 = pl.multiple_of(
            lax.shift_left(lax.shift_right_logical(r, 7), 7), BLK)
        copies.append(pltpu.async_copy(
            tab.at[:, pl.ds(col0, BLK)],
            blk.at[pl.ds(j * EMBED, EMBED), :], sem))
      for cp in copies:
        cp.wait()
      for v in range(CH // L):
        cols = lax.bitwise_and(vecs[v], lanes_mask)
        for j in range(L):
          el = v * L + j
          rows = iot + el * EMBED
          col = jnp.broadcast_to(cols[j], (L,))
          vec = plsc.load_gather(blk, [rows, col])
          rows_ref[pl.ds((c0 + el) * EMBED, EMBED)] = vec

    def u_round(c, carry):
      fetch_round(u_t, uidx, urows, c)
      return carry

    def i_round(c, carry):
      fetch_round(i_t, iidx, irows, c)
      return carry

    lax.fori_loop(0, n_rounds, u_round, 0)
    lax.fori_loop(0, n_rounds, i_round, 0)

    bias_vec = bv[...]
    wsp = [wv[e] for e in range(EMBED)]
    iot16 = iot * EMBED
    zero = jnp.zeros((L,), jnp.float32)

    def group(g, carry):
      g16 = pl.multiple_of(g * L, L)
      base_i = iot16 + g16 * EMBED
      accs = [bias_vec, zero, zero, zero]
      for e in range(EMBED):
        idx = base_i + e
        uc = plsc.load_gather(urows, [idx])
        ic = plsc.load_gather(irows, [idx])
        accs[e % 4] = accs[e % 4] + (uc * ic) * wsp[e]
      outv[pl.ds(g16, L)] = (accs[0] + accs[1]) + (accs[2] + accs[3])
      return carry

    lax.fori_loop(0, n_groups, group, 0)

    pltpu.sync_copy(outv, out_h.at[pl.ds(base, b_per_w)])

  return gmf


def kernel(user, item, U, I, fc_w, fc_b):
  B = user.shape[0]
  u_t = U.T
  i_t = I.T
  user1 = user.astype(jnp.int32)
  item1 = item.astype(jnp.int32)
  w_vec = jnp.broadcast_to(
      fc_w.reshape(EMBED, 1).astype(jnp.float32), (EMBED, L))
  b_vec = jnp.broadcast_to(fc_b.reshape(()), (L,)).astype(jnp.float32)
  return _build_sc_call(B)(user1, item1, u_t, i_t, w_vec, b_vec)
